# Initial kernel scaffold; baseline (speedup 1.0000x reference)
import jax
import jax.numpy as jnp
from jax.experimental import pallas as pl


def kernel(x_product, x_demographic, x_platform, edge_index_targets, edge_index_rev_targets, edge_index_uses, edge_index_rev_uses, edge_index_self, Wl_t, bl_t, Wr_t, Wl_rt, bl_rt, Wr_rt, Wl_u, bl_u, Wr_u, Wl_ru, bl_ru, Wr_ru, Wl_s, bl_s, Wr_s, W_lin, b_lin):
    raise NotImplementedError("write your pallas kernel here")


# speedup vs baseline: 7.5235x; 7.5235x over previous
import functools

import jax
import jax.numpy as jnp
from jax import lax
from jax.experimental import pallas as pl
from jax.experimental.pallas import tpu as pltpu
from jax.experimental.pallas import tpu_sc as plsc

N_PROD = 50000
D_IN = 32
D_H = 64
N_OUT = 100
N_EDGE = 800000
CHUNK = 128
N_CHUNKS = N_EDGE // CHUNK
N_SUB = 16
N_PAD = 50048
ROWS_PER_TILE = N_PAD // N_SUB


def _seg_body(src_rt, dst_rt, x_dem, src_s, dst_s, x_prod, zeros32, zeros8,
              ones_h,
              sum_rt, cnt_rt, sum_s, cnt_s,
              acc, cnt, idx_s, idx_d, rows, ones_v, sem):
    cid = lax.axis_index("c")
    sid = lax.axis_index("s")

    pltpu.sync_copy(zeros32, acc.at[pl.ds(sid * ROWS_PER_TILE, ROWS_PER_TILE)])
    pltpu.sync_copy(zeros8, cnt.at[pl.ds(sid * ROWS_PER_TILE, ROWS_PER_TILE)])
    pltpu.sync_copy(ones_h, ones_v)
    plsc.subcore_barrier()

    def do_relation(src_hbm, dst_hbm, xsrc_hbm):
        n_iter = (N_CHUNKS - sid + N_SUB - 1) // N_SUB

        def body(i, carry):
            base = (sid + i * N_SUB) * CHUNK
            pltpu.sync_copy(src_hbm.at[pl.ds(base, CHUNK)], idx_s)
            pltpu.sync_copy(dst_hbm.at[pl.ds(base, CHUNK)], idx_d)
            pltpu.async_copy(xsrc_hbm.at[idx_s], rows, sem).wait()
            pltpu.sync_copy(rows, acc.at[idx_d], add=True)
            pltpu.sync_copy(ones_v, cnt.at[idx_d], add=True)
            return carry

        lax.fori_loop(0, n_iter, body, 0)

    @pl.when(cid == 0)
    def _():
        do_relation(src_rt, dst_rt, x_dem)

    @pl.when(cid == 1)
    def _():
        do_relation(src_s, dst_s, x_prod)

    plsc.subcore_barrier()

    row0 = sid * ROWS_PER_TILE

    @pl.when(cid == 0)
    def _():
        pltpu.sync_copy(acc.at[pl.ds(row0, ROWS_PER_TILE)],
                        sum_rt.at[pl.ds(row0, ROWS_PER_TILE)])
        pltpu.sync_copy(cnt.at[pl.ds(row0, ROWS_PER_TILE)],
                        cnt_rt.at[pl.ds(row0, ROWS_PER_TILE)])

    @pl.when(cid == 1)
    def _():
        pltpu.sync_copy(acc.at[pl.ds(row0, ROWS_PER_TILE)],
                        sum_s.at[pl.ds(row0, ROWS_PER_TILE)])
        pltpu.sync_copy(cnt.at[pl.ds(row0, ROWS_PER_TILE)],
                        cnt_s.at[pl.ds(row0, ROWS_PER_TILE)])


@functools.partial(jax.jit, static_argnames=())
def _segment_sums(src_rt, dst_rt, x_dem, src_s, dst_s, x_prod):
    zeros32 = jnp.zeros((ROWS_PER_TILE, D_IN), jnp.float32)
    zeros8 = jnp.zeros((ROWS_PER_TILE, 8), jnp.float32)
    ones_h = jnp.ones((CHUNK, 8), jnp.float32)
    mesh = plsc.VectorSubcoreMesh(core_axis_name="c", subcore_axis_name="s")
    f = pl.kernel(
        _seg_body,
        out_type=[
            jax.ShapeDtypeStruct((N_PAD, D_IN), jnp.float32),
            jax.ShapeDtypeStruct((N_PAD, 8), jnp.float32),
            jax.ShapeDtypeStruct((N_PAD, D_IN), jnp.float32),
            jax.ShapeDtypeStruct((N_PAD, 8), jnp.float32),
        ],
        mesh=mesh,
        scratch_types=[
            pltpu.VMEM_SHARED((N_PAD, D_IN), jnp.float32),
            pltpu.VMEM_SHARED((N_PAD, 8), jnp.float32),
            pltpu.VMEM((CHUNK,), jnp.int32),
            pltpu.VMEM((CHUNK,), jnp.int32),
            pltpu.VMEM((CHUNK, D_IN), jnp.float32),
            pltpu.VMEM((CHUNK, 8), jnp.float32),
            pltpu.SemaphoreType.DMA,
        ],
        compiler_params=pltpu.CompilerParams(use_tc_tiling_on_sc=False),
    )
    return f(src_rt, dst_rt, x_dem, src_s, dst_s, x_prod, zeros32, zeros8,
             ones_h)


def _dense_body(sum_rt, cnt_rt, sum_s, cnt_s, xp,
                wl_rt, wl_s, wr_rt, wr_s, bl_rt, bl_s, wlin, blin, out):
    c1 = jnp.maximum(cnt_rt[:, 0:1], 1.0)
    c2 = jnp.maximum(cnt_s[:, 0:1], 1.0)
    m1 = sum_rt[...] / c1
    m2 = sum_s[...] / c2
    h = jnp.dot(m1, wl_rt[...], preferred_element_type=jnp.float32,
                precision=lax.Precision.HIGHEST)
    h = h + jnp.dot(m2, wl_s[...], preferred_element_type=jnp.float32,
                    precision=lax.Precision.HIGHEST)
    h = h + jnp.dot(xp[...], wr_rt[...] + wr_s[...],
                    preferred_element_type=jnp.float32,
                    precision=lax.Precision.HIGHEST)
    h = (h + (bl_rt[...] + bl_s[...])) * 0.5
    h = jnp.where(h >= 0, h, 0.01 * h)
    out[...] = jnp.dot(h, wlin[...], preferred_element_type=jnp.float32,
                       precision=lax.Precision.HIGHEST) + blin[...]


def _dense(sum_rt, cnt_rt, sum_s, cnt_s, xp, wl_rt, wl_s, wr_rt, wr_s,
           bl_rt, bl_s, wlin, blin):
    blk = 2000
    grid = (N_PROD // blk,)
    row_spec = lambda w: pl.BlockSpec((blk, w), lambda i: (i, 0))
    full = lambda a, b: pl.BlockSpec((a, b), lambda i: (0, 0))
    return pl.pallas_call(
        _dense_body,
        grid=grid,
        in_specs=[
            row_spec(D_IN), row_spec(8), row_spec(D_IN), row_spec(8),
            row_spec(D_IN),
            full(D_IN, D_H), full(D_IN, D_H), full(D_IN, D_H), full(D_IN, D_H),
            full(1, D_H), full(1, D_H), full(D_H, N_OUT), full(1, N_OUT),
        ],
        out_specs=row_spec(N_OUT),
        out_shape=jax.ShapeDtypeStruct((N_PROD, N_OUT), jnp.float32),
    )(sum_rt, cnt_rt, sum_s, cnt_s, xp, wl_rt, wl_s, wr_rt, wr_s,
      bl_rt.reshape(1, D_H), bl_s.reshape(1, D_H), wlin,
      blin.reshape(1, N_OUT))


def kernel(x_product, x_demographic, x_platform, edge_index_targets,
           edge_index_rev_targets, edge_index_uses, edge_index_rev_uses,
           edge_index_self,
           Wl_t, bl_t, Wr_t,
           Wl_rt, bl_rt, Wr_rt,
           Wl_u, bl_u, Wr_u,
           Wl_ru, bl_ru, Wr_ru,
           Wl_s, bl_s, Wr_s,
           W_lin, b_lin):
    src_rt = edge_index_rev_targets[0]
    dst_rt = edge_index_rev_targets[1]
    src_s = edge_index_self[0]
    dst_s = edge_index_self[1]
    sum_rt, cnt_rt, sum_s, cnt_s = _segment_sums(
        src_rt, dst_rt, x_demographic, src_s, dst_s, x_product)
    return _dense(sum_rt, cnt_rt, sum_s, cnt_s, x_product,
                  Wl_rt, Wl_s, Wr_rt, Wr_s, bl_rt, bl_s, W_lin, b_lin)


# speedup vs baseline: 10.6879x; 1.4206x over previous
import functools

import jax
import jax.numpy as jnp
from jax import lax
from jax.experimental import pallas as pl
from jax.experimental.pallas import tpu as pltpu
from jax.experimental.pallas import tpu_sc as plsc

N_PROD = 50000
D_IN = 32
D_H = 64
N_OUT = 100
N_EDGE = 800000
CHUNK = 128
N_SUB = 16
N_PAD = 50048
ROWS_PER_TILE = N_PAD // N_SUB
EDGE_ROWS = 6400
E_PAD = EDGE_ROWS * CHUNK
ROWS_T = EDGE_ROWS // N_SUB
KPIPE = 2
NS = ROWS_T // KPIPE
KCNT = 16
NCNT = ROWS_T // KCNT


def _feat_body(src_rt, dst_rt, x_dem, src_s, dst_s, x_prod, zeros32,
               sum_rt, sum_s,
               acc, rows, idx_s, idx_d, sem):
    cid = lax.axis_index("c")
    sid = lax.axis_index("s")
    row0 = sid * ROWS_T
    arow0 = sid * ROWS_PER_TILE

    pltpu.sync_copy(zeros32, acc.at[pl.ds(arow0, ROWS_PER_TILE)])
    plsc.subcore_barrier()

    def run(src2d, dst2d, xsrc_hbm, out_hbm):
        pltpu.sync_copy(src2d.at[pl.ds(row0, KPIPE)],
                        idx_s.at[pl.ds(0, KPIPE)])
        pltpu.sync_copy(dst2d.at[pl.ds(row0, KPIPE)],
                        idx_d.at[pl.ds(0, KPIPE)])
        for j in range(KPIPE):
            pltpu.async_copy(xsrc_hbm.at[idx_s.at[j]], rows.at[j], sem)

        def body(i, carry):
            off = (i % 2) * KPIPE
            offn = KPIPE - off

            @pl.when(i < NS - 1)
            def _():
                r_next = row0 + (i + 1) * KPIPE
                pltpu.sync_copy(src2d.at[pl.ds(r_next, KPIPE)],
                                idx_s.at[pl.ds(offn, KPIPE)])
                pltpu.sync_copy(dst2d.at[pl.ds(r_next, KPIPE)],
                                idx_d.at[pl.ds(offn, KPIPE)])
                for j in range(KPIPE):
                    pltpu.async_copy(xsrc_hbm.at[idx_s.at[offn + j]],
                                     rows.at[offn + j], sem)

            for j in range(KPIPE):
                pltpu.make_async_copy(xsrc_hbm.at[idx_s.at[off + j]],
                                      rows.at[off + j], sem).wait()
            for j in range(KPIPE):
                pltpu.sync_copy(rows.at[off + j],
                                acc.at[idx_d.at[off + j]], add=True)
            return carry

        lax.fori_loop(0, NS, body, 0)
        plsc.subcore_barrier()
        pltpu.sync_copy(acc.at[pl.ds(arow0, ROWS_PER_TILE)],
                        out_hbm.at[pl.ds(arow0, ROWS_PER_TILE)])

    @pl.when(cid == 0)
    def _():
        run(src_rt, dst_rt, x_dem, sum_rt)

    @pl.when(cid == 1)
    def _():
        run(src_s, dst_s, x_prod, sum_s)


def _cnt_body(dst_rt, dst_s, zeros8, ones_h,
              cnt_rt, cnt_s,
              cnt, idx_d, ones_v, sem):
    cid = lax.axis_index("c")
    sid = lax.axis_index("s")
    row0 = sid * ROWS_T
    arow0 = sid * ROWS_PER_TILE

    pltpu.sync_copy(zeros8, cnt.at[pl.ds(arow0, ROWS_PER_TILE)])
    pltpu.sync_copy(ones_h, ones_v)
    plsc.subcore_barrier()

    def run(dst2d, out_hbm):
        pltpu.sync_copy(dst2d.at[pl.ds(row0, KCNT)], idx_d.at[pl.ds(0, KCNT)])

        def body(i, carry):
            off = (i % 2) * KCNT
            offn = KCNT - off

            @pl.when(i < NCNT - 1)
            def _():
                pltpu.async_copy(dst2d.at[pl.ds(row0 + (i + 1) * KCNT, KCNT)],
                                 idx_d.at[pl.ds(offn, KCNT)], sem)

            for j in range(KCNT):
                pltpu.sync_copy(ones_v, cnt.at[idx_d.at[off + j]], add=True)

            @pl.when(i < NCNT - 1)
            def _():
                pltpu.make_async_copy(
                    dst2d.at[pl.ds(row0, KCNT)],
                    idx_d.at[pl.ds(offn, KCNT)], sem).wait()
            return carry

        lax.fori_loop(0, NCNT, body, 0)
        plsc.subcore_barrier()
        pltpu.sync_copy(cnt.at[pl.ds(arow0, ROWS_PER_TILE)],
                        out_hbm.at[pl.ds(arow0, ROWS_PER_TILE)])

    @pl.when(cid == 0)
    def _():
        run(dst_rt, cnt_rt)

    @pl.when(cid == 1)
    def _():
        run(dst_s, cnt_s)


@functools.partial(jax.jit, static_argnames=())
def _segment_sums(src_rt, dst_rt, x_dem, src_s, dst_s, x_prod):
    zeros32 = jnp.zeros((ROWS_PER_TILE, D_IN), jnp.float32)
    zeros8 = jnp.zeros((ROWS_PER_TILE, 8), jnp.float32)
    ones_h = jnp.ones((CHUNK, 8), jnp.float32)
    mesh = plsc.VectorSubcoreMesh(core_axis_name="c", subcore_axis_name="s")
    feat = pl.kernel(
        _feat_body,
        out_type=[
            jax.ShapeDtypeStruct((N_PAD, D_IN), jnp.float32),
            jax.ShapeDtypeStruct((N_PAD, D_IN), jnp.float32),
        ],
        mesh=mesh,
        scratch_types=[
            pltpu.VMEM_SHARED((N_PAD, D_IN), jnp.float32),
            pltpu.VMEM((2 * KPIPE, CHUNK, D_IN), jnp.float32),
            pltpu.VMEM((2 * KPIPE, CHUNK), jnp.int32),
            pltpu.VMEM((2 * KPIPE, CHUNK), jnp.int32),
            pltpu.SemaphoreType.DMA,
        ],
        compiler_params=pltpu.CompilerParams(use_tc_tiling_on_sc=False),
    )
    sum_rt, sum_s = feat(src_rt, dst_rt, x_dem, src_s, dst_s, x_prod, zeros32)
    cntk = pl.kernel(
        _cnt_body,
        out_type=[
            jax.ShapeDtypeStruct((N_PAD, 8), jnp.float32),
            jax.ShapeDtypeStruct((N_PAD, 8), jnp.float32),
        ],
        mesh=mesh,
        scratch_types=[
            pltpu.VMEM_SHARED((N_PAD, 8), jnp.float32),
            pltpu.VMEM((2 * KCNT, CHUNK), jnp.int32),
            pltpu.VMEM((CHUNK, 8), jnp.float32),
            pltpu.SemaphoreType.DMA,
        ],
        compiler_params=pltpu.CompilerParams(use_tc_tiling_on_sc=False),
    )
    cnt_rt, cnt_s = cntk(dst_rt, dst_s, zeros8, ones_h)
    return sum_rt, cnt_rt, sum_s, cnt_s


def _dense_body(sum_rt, cnt_rt, sum_s, cnt_s, xp,
                wl_rt, wl_s, wr_rt, wr_s, bl_rt, bl_s, wlin, blin, out):
    c1 = jnp.maximum(cnt_rt[:, 0:1], 1.0)
    c2 = jnp.maximum(cnt_s[:, 0:1], 1.0)
    m1 = sum_rt[...] / c1
    m2 = sum_s[...] / c2
    h = jnp.dot(m1, wl_rt[...], preferred_element_type=jnp.float32,
                precision=lax.Precision.HIGHEST)
    h = h + jnp.dot(m2, wl_s[...], preferred_element_type=jnp.float32,
                    precision=lax.Precision.HIGHEST)
    h = h + jnp.dot(xp[...], wr_rt[...] + wr_s[...],
                    preferred_element_type=jnp.float32,
                    precision=lax.Precision.HIGHEST)
    h = (h + (bl_rt[...] + bl_s[...])) * 0.5
    h = jnp.where(h >= 0, h, 0.01 * h)
    out[...] = jnp.dot(h, wlin[...], preferred_element_type=jnp.float32,
                       precision=lax.Precision.HIGHEST) + blin[...]


def _dense(sum_rt, cnt_rt, sum_s, cnt_s, xp, wl_rt, wl_s, wr_rt, wr_s,
           bl_rt, bl_s, wlin, blin):
    blk = 2000
    grid = (N_PROD // blk,)
    row_spec = lambda w: pl.BlockSpec((blk, w), lambda i: (i, 0))
    full = lambda a, b: pl.BlockSpec((a, b), lambda i: (0, 0))
    return pl.pallas_call(
        _dense_body,
        grid=grid,
        in_specs=[
            row_spec(D_IN), row_spec(8), row_spec(D_IN), row_spec(8),
            row_spec(D_IN),
            full(D_IN, D_H), full(D_IN, D_H), full(D_IN, D_H), full(D_IN, D_H),
            full(1, D_H), full(1, D_H), full(D_H, N_OUT), full(1, N_OUT),
        ],
        out_specs=row_spec(N_OUT),
        out_shape=jax.ShapeDtypeStruct((N_PROD, N_OUT), jnp.float32),
    )(sum_rt, cnt_rt, sum_s, cnt_s, xp, wl_rt, wl_s, wr_rt, wr_s,
      bl_rt.reshape(1, D_H), bl_s.reshape(1, D_H), wlin,
      blin.reshape(1, N_OUT))


def kernel(x_product, x_demographic, x_platform, edge_index_targets,
           edge_index_rev_targets, edge_index_uses, edge_index_rev_uses,
           edge_index_self,
           Wl_t, bl_t, Wr_t,
           Wl_rt, bl_rt, Wr_rt,
           Wl_u, bl_u, Wr_u,
           Wl_ru, bl_ru, Wr_ru,
           Wl_s, bl_s, Wr_s,
           W_lin, b_lin):
    def pad2d(a, fill):
        pad = jnp.full((E_PAD - N_EDGE,), fill, jnp.int32)
        return jnp.concatenate([a, pad]).reshape(EDGE_ROWS, CHUNK)

    src_rt = pad2d(edge_index_rev_targets[0], 0)
    dst_rt = pad2d(edge_index_rev_targets[1], N_PROD)
    src_s = pad2d(edge_index_self[0], 0)
    dst_s = pad2d(edge_index_self[1], N_PROD)
    sum_rt, cnt_rt, sum_s, cnt_s = _segment_sums(
        src_rt, dst_rt, x_demographic, src_s, dst_s, x_product)
    return _dense(sum_rt, cnt_rt, sum_s, cnt_s, x_product,
                  Wl_rt, Wl_s, Wr_rt, Wr_s, bl_rt, bl_s, W_lin, b_lin)


# speedup vs baseline: 12.0548x; 1.1279x over previous
import functools

import jax
import jax.numpy as jnp
from jax import lax
from jax.experimental import pallas as pl
from jax.experimental.pallas import tpu as pltpu
from jax.experimental.pallas import tpu_sc as plsc

N_PROD = 50000
D_IN = 32
D_H = 64
N_OUT = 100
N_EDGE = 800000
CHUNK = 128
N_SUB = 16
N_PAD = 50048
ROWS_PER_TILE = N_PAD // N_SUB
EDGE_ROWS = 6400
E_PAD = EDGE_ROWS * CHUNK
ROWS_T = EDGE_ROWS // N_SUB
KPIPE = 2
NS = ROWS_T // KPIPE
KCNT = 16
NCNT = ROWS_T // KCNT


RING = 5
IBLK = 8
IHALVES = 3
NBLK = ROWS_T // IBLK


def _feat_body(src_rt, dst_rt, x_dem, src_s, dst_s, x_prod, zeros32,
               sum_rt, sum_s,
               acc, rows, idx_s, idx_d, semG, semA, semI):
    cid = lax.axis_index("c")
    sid = lax.axis_index("s")
    row0 = sid * ROWS_T
    arow0 = sid * ROWS_PER_TILE

    pltpu.sync_copy(zeros32, acc.at[pl.ds(arow0, ROWS_PER_TILE)])
    plsc.subcore_barrier()

    def run(src2d, dst2d, xsrc_hbm, out_hbm):
        def idx_load(blk):
            half = (blk % IHALVES) * IBLK
            pltpu.async_copy(src2d.at[pl.ds(row0 + blk * IBLK, IBLK)],
                             idx_s.at[pl.ds(half, IBLK)], semI)
            pltpu.async_copy(dst2d.at[pl.ds(row0 + blk * IBLK, IBLK)],
                             idx_d.at[pl.ds(half, IBLK)], semI)

        def idx_wait():
            pltpu.make_async_copy(src2d.at[pl.ds(row0, IBLK)],
                                  idx_s.at[pl.ds(0, IBLK)], semI).wait()
            pltpu.make_async_copy(dst2d.at[pl.ds(row0, IBLK)],
                                  idx_d.at[pl.ds(0, IBLK)], semI).wait()

        def add_wait():
            pltpu.make_async_copy(rows.at[0], acc.at[idx_d.at[0]],
                                  semA).wait()

        idx_load(0)
        idx_load(1)
        idx_wait()
        for j in range(RING - 2):
            pltpu.async_copy(xsrc_hbm.at[idx_s.at[j]], rows.at[j], semG)

        def body(c, carry):
            crow = c % (IHALVES * IBLK)
            slot = c % RING
            pltpu.make_async_copy(xsrc_hbm.at[idx_s.at[crow]],
                                  rows.at[slot], semG).wait()
            pltpu.async_copy(rows.at[slot], acc.at[idx_d.at[crow]], semA,
                             add=True)

            @pl.when(c >= 2)
            def _():
                add_wait()

            @pl.when((c % IBLK == IBLK - 3) & (c + RING - 2 < ROWS_T))
            def _():
                idx_wait()

            @pl.when(c + RING - 2 < ROWS_T)
            def _():
                n = c + RING - 2
                pltpu.async_copy(xsrc_hbm.at[idx_s.at[n % (IHALVES * IBLK)]],
                                 rows.at[n % RING], semG)

            @pl.when((c % IBLK == 2) & (c // IBLK + 2 < NBLK))
            def _():
                idx_load(c // IBLK + 2)

            return carry

        lax.fori_loop(0, ROWS_T, body, 0)
        add_wait()
        add_wait()
        plsc.subcore_barrier()
        pltpu.sync_copy(acc.at[pl.ds(arow0, ROWS_PER_TILE)],
                        out_hbm.at[pl.ds(arow0, ROWS_PER_TILE)])

    @pl.when(cid == 0)
    def _():
        run(src_rt, dst_rt, x_dem, sum_rt)

    @pl.when(cid == 1)
    def _():
        run(src_s, dst_s, x_prod, sum_s)


def _cnt_body(dst_rt, dst_s, zeros8, ones_h,
              cnt_rt, cnt_s,
              cnt, idx_d, ones_v, sem):
    cid = lax.axis_index("c")
    sid = lax.axis_index("s")
    row0 = sid * ROWS_T
    arow0 = sid * ROWS_PER_TILE

    pltpu.sync_copy(zeros8, cnt.at[pl.ds(arow0, ROWS_PER_TILE)])
    pltpu.sync_copy(ones_h, ones_v)
    plsc.subcore_barrier()

    def run(dst2d, out_hbm):
        pltpu.sync_copy(dst2d.at[pl.ds(row0, KCNT)], idx_d.at[pl.ds(0, KCNT)])

        def body(i, carry):
            off = (i % 2) * KCNT
            offn = KCNT - off

            @pl.when(i < NCNT - 1)
            def _():
                pltpu.async_copy(dst2d.at[pl.ds(row0 + (i + 1) * KCNT, KCNT)],
                                 idx_d.at[pl.ds(offn, KCNT)], sem)

            for j in range(KCNT):
                pltpu.sync_copy(ones_v, cnt.at[idx_d.at[off + j]], add=True)

            @pl.when(i < NCNT - 1)
            def _():
                pltpu.make_async_copy(
                    dst2d.at[pl.ds(row0, KCNT)],
                    idx_d.at[pl.ds(offn, KCNT)], sem).wait()
            return carry

        lax.fori_loop(0, NCNT, body, 0)
        plsc.subcore_barrier()
        pltpu.sync_copy(cnt.at[pl.ds(arow0, ROWS_PER_TILE)],
                        out_hbm.at[pl.ds(arow0, ROWS_PER_TILE)])

    @pl.when(cid == 0)
    def _():
        run(dst_rt, cnt_rt)

    @pl.when(cid == 1)
    def _():
        run(dst_s, cnt_s)


@functools.partial(jax.jit, static_argnames=())
def _segment_sums(src_rt, dst_rt, x_dem, src_s, dst_s, x_prod):
    zeros32 = jnp.zeros((ROWS_PER_TILE, D_IN), jnp.float32)
    zeros8 = jnp.zeros((ROWS_PER_TILE, 8), jnp.float32)
    ones_h = jnp.ones((CHUNK, 8), jnp.float32)
    mesh = plsc.VectorSubcoreMesh(core_axis_name="c", subcore_axis_name="s")
    feat = pl.kernel(
        _feat_body,
        out_type=[
            jax.ShapeDtypeStruct((N_PAD, D_IN), jnp.float32),
            jax.ShapeDtypeStruct((N_PAD, D_IN), jnp.float32),
        ],
        mesh=mesh,
        scratch_types=[
            pltpu.VMEM_SHARED((N_PAD, D_IN), jnp.float32),
            pltpu.VMEM((RING, CHUNK, D_IN), jnp.float32),
            pltpu.VMEM((IHALVES * IBLK, CHUNK), jnp.int32),
            pltpu.VMEM((IHALVES * IBLK, CHUNK), jnp.int32),
            pltpu.SemaphoreType.DMA,
            pltpu.SemaphoreType.DMA,
            pltpu.SemaphoreType.DMA,
        ],
        compiler_params=pltpu.CompilerParams(use_tc_tiling_on_sc=False),
    )
    sum_rt, sum_s = feat(src_rt, dst_rt, x_dem, src_s, dst_s, x_prod, zeros32)
    cntk = pl.kernel(
        _cnt_body,
        out_type=[
            jax.ShapeDtypeStruct((N_PAD, 8), jnp.float32),
            jax.ShapeDtypeStruct((N_PAD, 8), jnp.float32),
        ],
        mesh=mesh,
        scratch_types=[
            pltpu.VMEM_SHARED((N_PAD, 8), jnp.float32),
            pltpu.VMEM((2 * KCNT, CHUNK), jnp.int32),
            pltpu.VMEM((CHUNK, 8), jnp.float32),
            pltpu.SemaphoreType.DMA,
        ],
        compiler_params=pltpu.CompilerParams(use_tc_tiling_on_sc=False),
    )
    cnt_rt, cnt_s = cntk(dst_rt, dst_s, zeros8, ones_h)
    return sum_rt, cnt_rt, sum_s, cnt_s


def _dense_body(sum_rt, cnt_rt, sum_s, cnt_s, xp,
                wl_rt, wl_s, wr_rt, wr_s, bl_rt, bl_s, wlin, blin, out):
    c1 = jnp.maximum(cnt_rt[:, 0:1], 1.0)
    c2 = jnp.maximum(cnt_s[:, 0:1], 1.0)
    m1 = sum_rt[...] / c1
    m2 = sum_s[...] / c2
    h = jnp.dot(m1, wl_rt[...], preferred_element_type=jnp.float32,
                precision=lax.Precision.HIGHEST)
    h = h + jnp.dot(m2, wl_s[...], preferred_element_type=jnp.float32,
                    precision=lax.Precision.HIGHEST)
    h = h + jnp.dot(xp[...], wr_rt[...] + wr_s[...],
                    preferred_element_type=jnp.float32,
                    precision=lax.Precision.HIGHEST)
    h = (h + (bl_rt[...] + bl_s[...])) * 0.5
    h = jnp.where(h >= 0, h, 0.01 * h)
    out[...] = jnp.dot(h, wlin[...], preferred_element_type=jnp.float32,
                       precision=lax.Precision.HIGHEST) + blin[...]


def _dense(sum_rt, cnt_rt, sum_s, cnt_s, xp, wl_rt, wl_s, wr_rt, wr_s,
           bl_rt, bl_s, wlin, blin):
    blk = 2000
    grid = (N_PROD // blk,)
    row_spec = lambda w: pl.BlockSpec((blk, w), lambda i: (i, 0))
    full = lambda a, b: pl.BlockSpec((a, b), lambda i: (0, 0))
    return pl.pallas_call(
        _dense_body,
        grid=grid,
        in_specs=[
            row_spec(D_IN), row_spec(8), row_spec(D_IN), row_spec(8),
            row_spec(D_IN),
            full(D_IN, D_H), full(D_IN, D_H), full(D_IN, D_H), full(D_IN, D_H),
            full(1, D_H), full(1, D_H), full(D_H, N_OUT), full(1, N_OUT),
        ],
        out_specs=row_spec(N_OUT),
        out_shape=jax.ShapeDtypeStruct((N_PROD, N_OUT), jnp.float32),
    )(sum_rt, cnt_rt, sum_s, cnt_s, xp, wl_rt, wl_s, wr_rt, wr_s,
      bl_rt.reshape(1, D_H), bl_s.reshape(1, D_H), wlin,
      blin.reshape(1, N_OUT))


def kernel(x_product, x_demographic, x_platform, edge_index_targets,
           edge_index_rev_targets, edge_index_uses, edge_index_rev_uses,
           edge_index_self,
           Wl_t, bl_t, Wr_t,
           Wl_rt, bl_rt, Wr_rt,
           Wl_u, bl_u, Wr_u,
           Wl_ru, bl_ru, Wr_ru,
           Wl_s, bl_s, Wr_s,
           W_lin, b_lin):
    def pad2d(a, fill):
        pad = jnp.full((E_PAD - N_EDGE,), fill, jnp.int32)
        return jnp.concatenate([a, pad]).reshape(EDGE_ROWS, CHUNK)

    src_rt = pad2d(edge_index_rev_targets[0], 0)
    dst_rt = pad2d(edge_index_rev_targets[1], N_PROD)
    src_s = pad2d(edge_index_self[0], 0)
    dst_s = pad2d(edge_index_self[1], N_PROD)
    sum_rt, cnt_rt, sum_s, cnt_s = _segment_sums(
        src_rt, dst_rt, x_demographic, src_s, dst_s, x_product)
    return _dense(sum_rt, cnt_rt, sum_s, cnt_s, x_product,
                  Wl_rt, Wl_s, Wr_rt, Wr_s, bl_rt, bl_s, W_lin, b_lin)
